# baseline (speedup 1.0000x reference)
import functools

import jax
import jax.numpy as jnp
from jax import lax
from jax.experimental import pallas as pl
from jax.experimental.pallas import tpu as pltpu
from jax.experimental.pallas import tpu_sc as plsc

D = 768
L = 16
G = D // L
EPS = 1e-5
_RSQRT_SEED = 0x5F3759DF


def _rsqrt16(x):
    i = lax.bitcast_convert_type(x, jnp.int32)
    seed = jnp.full((L,), _RSQRT_SEED, jnp.int32)
    y = lax.bitcast_convert_type(seed - (i >> 1), jnp.float32)
    half_x = 0.5 * x
    for _ in range(4):
        y = y * (1.5 - half_x * y * y)
    return y


def _make_kernel(N, S, C):
    info = plsc.get_sparse_core_info()
    NC, NS = info.num_cores, info.num_subcores
    NW = NC * NS
    assert N % NW == 0
    TPW = N // NW
    assert TPW % C == 0 and S % TPW == 0
    CHUNKS = TPW // C

    mesh = plsc.VectorSubcoreMesh(core_axis_name="c", subcore_axis_name="s")

    @functools.partial(
        pl.kernel,
        out_type=jax.ShapeDtypeStruct((N, D), jnp.float32),
        mesh=mesh,
        compiler_params=pltpu.CompilerParams(needs_layout_passes=False),
        scratch_types=[
            pltpu.VMEM((C,), jnp.int32),
            pltpu.VMEM((C,), jnp.int32),
            pltpu.VMEM((C, D), jnp.float32),
            pltpu.VMEM((C, D), jnp.float32),
            pltpu.VMEM((2, D), jnp.float32),
            pltpu.VMEM((D,), jnp.float32),
            pltpu.VMEM((D,), jnp.float32),
            pltpu.SemaphoreType.DMA,
        ],
    )
    def k(x_hbm, seg_hbm, tok_hbm, pos_hbm, segtab_hbm, gamma_hbm, beta_hbm,
          out_hbm, idx_v, segi_v, tokbuf, posbuf, segtab_v, gamma_v, beta_v,
          sem):
        wid = lax.axis_index("s") * NC + lax.axis_index("c")
        base = wid * TPW

        pltpu.sync_copy(segtab_hbm, segtab_v)
        pltpu.sync_copy(gamma_hbm, gamma_v)
        pltpu.sync_copy(beta_hbm, beta_v)

        def chunk_body(c, carry):
            cb = base + c * C
            pltpu.sync_copy(x_hbm.at[pl.ds(cb, C)], idx_v)
            pltpu.sync_copy(seg_hbm.at[pl.ds(cb, C)], segi_v)
            pltpu.async_copy(tok_hbm.at[idx_v], tokbuf, sem).wait()
            pos0 = lax.rem(cb, S)
            pltpu.sync_copy(pos_hbm.at[pl.ds(pos0, C)], posbuf)

            def tok_body(i, carry2):
                sub = lax.rem(i, L)
                v16 = segi_v[pl.ds(i - sub, L)].astype(jnp.float32)
                sel = jnp.where(lax.iota(jnp.int32, L) == sub, v16, 0.0)
                segf = jnp.full((L,), jnp.sum(sel))
                acc = jnp.zeros((L,), jnp.float32)
                acq = jnp.zeros((L,), jnp.float32)
                for j in range(G):
                    sl = pl.ds(j * L, L)
                    s0 = segtab_v[0, sl]
                    s1 = segtab_v[1, sl]
                    t = tokbuf[i, sl] + posbuf[i, sl] + (s0 + segf * (s1 - s0))
                    tokbuf[i, sl] = t
                    acc = acc + t
                    acq = acq + t * t
                tot = jnp.sum(acc)
                sq = jnp.sum(acq)
                mean = tot * (1.0 / D)
                var = sq * (1.0 / D) - mean * mean
                mean_v = jnp.full((L,), mean, jnp.float32)
                rs_v = _rsqrt16(jnp.full((L,), var + EPS, jnp.float32))
                for j in range(G):
                    sl = pl.ds(j * L, L)
                    o = (tokbuf[i, sl] - mean_v) * rs_v
                    tokbuf[i, sl] = o * gamma_v[sl] + beta_v[sl]
                return carry2

            lax.fori_loop(0, C, tok_body, 0)
            pltpu.sync_copy(tokbuf, out_hbm.at[pl.ds(cb, C)])
            return carry

        lax.fori_loop(0, CHUNKS, chunk_body, 0)

    return k


def kernel(x, seg, tok_table, pos_table, seg_table, gamma, beta):
    B, S = x.shape
    N = B * S
    k = _make_kernel(N, S, C=64)
    out = k(x.reshape(N), seg.reshape(N), tok_table, pos_table, seg_table,
            gamma, beta)
    return out.reshape(B, S, D)
